# baseline (speedup 1.0000x reference)
import functools

import jax
import jax.numpy as jnp
from jax import lax
from jax.experimental import pallas as pl
from jax.experimental.pallas import tpu as pltpu
from jax.experimental.pallas import tpu_sc as plsc

B = 32; N = 512; D = 16; H = 128; WV = 300
L = 3; EDIM = 16; ELD = 32; ENT = 2; ES = 8; REL = 32
E_PB = N * D
CH = 64
NCH = E_PB // CH
NODES_PER_CH = CH // D
NBUF = 4
BG = B // 2

f32 = jnp.float32

_info = plsc.get_sparse_core_info()
_NC, _NS = _info.num_cores, _info.num_subcores
NW = _NC * _NS


def _edge_table(h, We1_ref, edge_emb_ref, We2_ref, b_edge_ref):
    E2 = (jnp.dot(edge_emb_ref[...], We2_ref[...], preferred_element_type=f32)
          + b_edge_ref[...])
    u = jnp.dot(h, We1_ref[...], preferred_element_type=f32)
    return jnp.tanh(u[None, :, :] + E2[:, None, :])


def _gates(h, c, rin, rout, Wcat_l, bcat_l):
    cat = jnp.concatenate([rin, rout], axis=1)
    z = jnp.dot(cat, Wcat_l, preferred_element_type=f32) + bcat_l
    ig = jax.nn.sigmoid(z[:, 0:H])
    og = jax.nn.sigmoid(z[:, H:2 * H])
    fg = jax.nn.sigmoid(z[:, 2 * H:3 * H])
    g = jnp.tanh(z[:, 3 * H:4 * H])
    c2 = fg * c + ig * g
    return og * jnp.tanh(c2), c2


def _body_A(lemmas_ref, in_nodes_ref, in_labels_ref, out_nodes_ref,
            out_labels_ref, W_word_ref, b_word_ref, edge_emb_ref,
            We1_ref, We2_ref, b_edge_ref,
            h_ref, t_ref, gin_ref, gout_ref):
    b = pl.program_id(0)
    lem = lemmas_ref[0]
    h = jnp.tanh(jnp.dot(lem, W_word_ref[...], preferred_element_type=f32)
                 + b_word_ref[...])
    h_ref[0] = h
    t_ref[0] = _edge_table(h, We1_ref, edge_emb_ref, We2_ref, b_edge_ref)
    off = b * E_PB
    gin_ref[0] = in_labels_ref[0] * N + in_nodes_ref[0] + off
    gout_ref[0] = out_labels_ref[0] * N + out_nodes_ref[0] + off


def _body_B(h_ref, c_ref, rio_ref, Wcat_ref, bcat_ref,
            edge_emb_ref, We1_ref, We2_ref, b_edge_ref,
            h2_ref, c2_ref, t_ref):
    h, c2 = _gates(h_ref[0], c_ref[0], rio_ref[0, 0], rio_ref[0, 1],
                   Wcat_ref[...], bcat_ref[...])
    h2_ref[0] = h
    c2_ref[0] = c2
    t_ref[0] = _edge_table(h, We1_ref, edge_emb_ref, We2_ref, b_edge_ref)


def _body_E(h_ref, c_ref, rio_ref, Wcat_ref, bcat_ref,
            ent_idx_ref, ent_mask_ref, W_rel_ref, b_rel_ref, out_ref):
    h, _ = _gates(h_ref[0], c_ref[0], rio_ref[0, 0], rio_ref[0, 1],
                  Wcat_ref[...], bcat_ref[...])
    eidx = ent_idx_ref[0]
    iota = lax.broadcasted_iota(jnp.int32, (ENT * ES, N), 1)
    ent_oh = (iota == eidx).astype(f32)
    ent_h = jnp.dot(ent_oh, h, preferred_element_type=f32)

    m = ent_mask_ref[0]
    r_io = lax.broadcasted_iota(jnp.int32, (ENT, ENT * ES), 0)
    e_io = lax.broadcasted_iota(jnp.int32, (ENT, ENT * ES), 1)
    P = ((e_io // ES) == r_io).astype(f32) * m
    denom = jnp.sum(P, axis=1, keepdims=True) + 1e-6
    ent_rep = jnp.dot(P, ent_h, preferred_element_type=f32) / denom
    flat = jnp.concatenate([ent_rep[0:1, :], ent_rep[1:2, :]], axis=1)
    out_ref[0] = jnp.dot(flat, W_rel_ref[...],
                         preferred_element_type=f32) + b_rel_ref[...]


_sc_mesh = plsc.VectorSubcoreMesh(core_axis_name="c", subcore_axis_name="s")


@functools.partial(
    pl.kernel, mesh=_sc_mesh,
    out_type=jax.ShapeDtypeStruct((BG * 2 * N * H,), f32),
    scratch_types=[
        pltpu.VMEM((NCH, CH), jnp.int32),
        pltpu.VMEM((NBUF, CH, H), f32),
        pltpu.VMEM((N * H,), f32),
    ] + [pltpu.SemaphoreType.DMA] * NBUF,
)
def _sc_gather(t_hbm, gin_hbm, gout_hbm, rio_hbm, idx_v, gbuf, rep_v, *sems):
    wid = lax.axis_index("s") * _NC + lax.axis_index("c")
    bat = wid >> 1
    d_out = (wid & 1) == 1

    def gather(ci, bslot):
        pltpu.make_async_copy(t_hbm.at[idx_v.at[ci]], gbuf.at[bslot],
                              sems[bslot]).start()

    def gather_wait(ci, bslot):
        pltpu.make_async_copy(t_hbm.at[idx_v.at[ci]], gbuf.at[bslot],
                              sems[bslot]).wait()

    def accum(ci, bslot):
        def node_pair(p, _):
            for half in range(2):
                n = p * 2 + half
                row0 = n * D
                for k in range(H // 16):
                    vals = [gbuf[bslot, row0 + j, pl.ds(k * 16, 16)]
                            for j in range(D)]
                    while len(vals) > 1:
                        vals = [vals[i] + vals[i + 1]
                                for i in range(0, len(vals), 2)]
                    rep_v[pl.ds((ci * NODES_PER_CH + n) * H + k * 16, 16)] = \
                        vals[0]
            return 0
        lax.fori_loop(0, NODES_PER_CH // 2, node_pair, 0)

    def run_dir(idx_hbm):
        pltpu.sync_copy(idx_hbm.at[pl.ds(bat * NCH, NCH)], idx_v)
        for b in range(NBUF):
            gather(b, b)

        def group_body(g, _):
            for b in range(NBUF):
                ci = g * NBUF + b
                gather_wait(ci, b)
                accum(ci, b)
                nxt = ci + NBUF

                @pl.when(nxt < NCH)
                def _():
                    gather(nxt, b)
            return 0

        lax.fori_loop(0, NCH // NBUF, group_body, 0)
        pltpu.sync_copy(
            rep_v, rio_hbm.at[pl.ds((bat * 2 + (wid & 1)) * N * H, N * H)])

    @pl.when(jnp.logical_not(d_out))
    def _():
        run_dir(gin_hbm)

    @pl.when(d_out)
    def _():
        run_dir(gout_hbm)


def _bspec(blk, im):
    return pl.BlockSpec(blk, im)


def _full(arr):
    return pl.BlockSpec(arr.shape, lambda b: (0,) * arr.ndim)


def kernel(node_num, lemmas, lemmas_idx, lemmas_chars, in_nodes, in_labels,
           out_nodes, out_labels, entity_indexs, truth_tags,
           in_node_mask, out_node_mask, entity_mask,
           W_word, b_word, edge_emb, W_edge, b_edge,
           w_in_i, w_out_i, b_i, w_in_o, w_out_o, b_o,
           w_in_f, w_out_f, b_f, w_in_cell, w_out_cell, b_cell,
           W_rel, b_rel):
    We1 = W_edge[:H]
    We2 = W_edge[H:]
    Wcat = jnp.concatenate([
        jnp.concatenate([w_in_i, w_in_o, w_in_f, w_in_cell], axis=2),
        jnp.concatenate([w_out_i, w_out_o, w_out_f, w_out_cell], axis=2),
    ], axis=1)
    bcat = jnp.concatenate([b_i, b_o, b_f, b_cell], axis=1)
    b_word2 = b_word.reshape(1, H)
    b_edge2 = b_edge.reshape(1, H)
    ent_idx = entity_indexs.reshape(B, ENT * ES, 1).astype(jnp.int32)
    ent_m = entity_mask.reshape(B, 1, ENT * ES)

    nhw = lambda: _bspec((1, N, H), lambda b: (b, 0, 0))
    rio_spec = lambda: _bspec((1, 2, N, H), lambda b: (b, 0, 0, 0))

    def run_A(lem_g, inn_g, inl_g, outn_g, outl_g):
        return pl.pallas_call(
            _body_A,
            grid=(BG,),
            in_specs=[
                _bspec((1, N, WV), lambda b: (b, 0, 0)),
                _bspec((1, N, D), lambda b: (b, 0, 0)),
                _bspec((1, N, D), lambda b: (b, 0, 0)),
                _bspec((1, N, D), lambda b: (b, 0, 0)),
                _bspec((1, N, D), lambda b: (b, 0, 0)),
                _full(W_word),
                _bspec((1, H), lambda b: (0, 0)),
                _full(edge_emb),
                _full(We1),
                _full(We2),
                _bspec((1, H), lambda b: (0, 0)),
            ],
            out_specs=[
                nhw(),
                _bspec((1, EDIM, N, H), lambda b: (b, 0, 0, 0)),
                _bspec((1, N, D), lambda b: (b, 0, 0)),
                _bspec((1, N, D), lambda b: (b, 0, 0)),
            ],
            out_shape=[
                jax.ShapeDtypeStruct((BG, N, H), f32),
                jax.ShapeDtypeStruct((BG, EDIM, N, H), f32),
                jax.ShapeDtypeStruct((BG, N, D), jnp.int32),
                jax.ShapeDtypeStruct((BG, N, D), jnp.int32),
            ],
        )(lem_g, inn_g, inl_g, outn_g, outl_g,
          W_word, b_word2, edge_emb, We1, We2, b_edge2)

    def run_B(h, c, rio, l):
        return pl.pallas_call(
            _body_B,
            grid=(BG,),
            in_specs=[nhw(), nhw(), rio_spec(),
                      _full(Wcat[l]), _bspec((1, 4 * H), lambda b: (0, 0)),
                      _full(edge_emb), _full(We1), _full(We2),
                      _bspec((1, H), lambda b: (0, 0))],
            out_specs=[nhw(), nhw(),
                       _bspec((1, EDIM, N, H), lambda b: (b, 0, 0, 0))],
            out_shape=[
                jax.ShapeDtypeStruct((BG, N, H), f32),
                jax.ShapeDtypeStruct((BG, N, H), f32),
                jax.ShapeDtypeStruct((BG, EDIM, N, H), f32),
            ],
        )(h, c, rio, Wcat[l], bcat[l].reshape(1, 4 * H),
          edge_emb, We1, We2, b_edge2)

    def run_E(h, c, rio, eidx_g, em_g):
        return pl.pallas_call(
            _body_E,
            grid=(BG,),
            in_specs=[nhw(), nhw(), rio_spec(),
                      _full(Wcat[L - 1]), _bspec((1, 4 * H), lambda b: (0, 0)),
                      _bspec((1, ENT * ES, 1), lambda b: (b, 0, 0)),
                      _bspec((1, 1, ENT * ES), lambda b: (b, 0, 0)),
                      _full(W_rel), _bspec((1, REL), lambda b: (0, 0))],
            out_specs=_bspec((1, 1, REL), lambda b: (b, 0, 0)),
            out_shape=jax.ShapeDtypeStruct((BG, 1, REL), f32),
        )(h, c, rio, Wcat[L - 1], bcat[L - 1].reshape(1, 4 * H),
          eidx_g, em_g, W_rel, b_rel.reshape(1, REL))

    def run_group(sl):
        h, T, gin, gout = run_A(lemmas[sl], in_nodes[sl], in_labels[sl],
                                out_nodes[sl], out_labels[sl])
        gin2 = gin.reshape(BG * NCH, CH)
        gout2 = gout.reshape(BG * NCH, CH)
        c = jnp.zeros((BG, N, H), f32)
        for l in range(L - 1):
            rio = _sc_gather(T.reshape(BG * E_PB, H), gin2, gout2)
            h, c, T = run_B(h, c, rio.reshape(BG, 2, N, H), l)
        rio = _sc_gather(T.reshape(BG * E_PB, H), gin2, gout2)
        return run_E(h, c, rio.reshape(BG, 2, N, H), ent_idx[sl], ent_m[sl])

    out0 = run_group(slice(0, BG))
    out1 = run_group(slice(BG, B))
    return jnp.concatenate([out0, out1], axis=0).reshape(B, REL)


# speedup vs baseline: 1.2316x; 1.2316x over previous
import functools

import jax
import jax.numpy as jnp
from jax import lax
from jax.experimental import pallas as pl
from jax.experimental.pallas import tpu as pltpu
from jax.experimental.pallas import tpu_sc as plsc

B = 32; N = 512; D = 16; H = 128; WV = 300
L = 3; EDIM = 16; ELD = 32; ENT = 2; ES = 8; REL = 32
E_PB = N * D
CH = 64
NCH = E_PB // CH
NODES_PER_CH = CH // D
NBUF = 4
BG = B // 2

f32 = jnp.float32

_info = plsc.get_sparse_core_info()
_NC, _NS = _info.num_cores, _info.num_subcores
NW = _NC * _NS


def _edge_table(h, We1_ref, edge_emb_ref, We2_ref, b_edge_ref):
    E2 = (jnp.dot(edge_emb_ref[...], We2_ref[...], preferred_element_type=f32)
          + b_edge_ref[...])
    u = jnp.dot(h, We1_ref[...], preferred_element_type=f32)
    return jnp.tanh(u[None, :, :] + E2[:, None, :])


def _gates(h, c, rin, rout, Wcat_l, bcat_l):
    cat = jnp.concatenate([rin, rout], axis=1)
    z = jnp.dot(cat, Wcat_l, preferred_element_type=f32) + bcat_l
    ig = jax.nn.sigmoid(z[:, 0:H])
    og = jax.nn.sigmoid(z[:, H:2 * H])
    fg = jax.nn.sigmoid(z[:, 2 * H:3 * H])
    g = jnp.tanh(z[:, 3 * H:4 * H])
    c2 = fg * c + ig * g
    return og * jnp.tanh(c2), c2


def _body_A(lemmas_ref, in_nodes_ref, in_labels_ref, out_nodes_ref,
            out_labels_ref, W_word_ref, b_word_ref, edge_emb_ref,
            We1_ref, We2_ref, b_edge_ref,
            h_ref, t_ref, gin_ref, gout_ref):
    b = pl.program_id(0)
    lem = lemmas_ref[0]
    h = jnp.tanh(jnp.dot(lem, W_word_ref[...], preferred_element_type=f32)
                 + b_word_ref[...])
    h_ref[0] = h
    t_ref[0] = _edge_table(h, We1_ref, edge_emb_ref, We2_ref, b_edge_ref)
    off = b * E_PB
    gin_ref[0] = in_labels_ref[0] * N + in_nodes_ref[0] + off
    gout_ref[0] = out_labels_ref[0] * N + out_nodes_ref[0] + off


def _body_B(h_ref, c_ref, rio_ref, Wcat_ref, bcat_ref,
            edge_emb_ref, We1_ref, We2_ref, b_edge_ref,
            h2_ref, c2_ref, t_ref):
    h, c2 = _gates(h_ref[0], c_ref[0], rio_ref[0, 0], rio_ref[0, 1],
                   Wcat_ref[...], bcat_ref[...])
    h2_ref[0] = h
    c2_ref[0] = c2
    t_ref[0] = _edge_table(h, We1_ref, edge_emb_ref, We2_ref, b_edge_ref)


def _body_E(h_ref, c_ref, rio_ref, Wcat_ref, bcat_ref,
            ent_idx_ref, ent_mask_ref, W_rel_ref, b_rel_ref, out_ref):
    h, _ = _gates(h_ref[0], c_ref[0], rio_ref[0, 0], rio_ref[0, 1],
                  Wcat_ref[...], bcat_ref[...])
    eidx = ent_idx_ref[0]
    iota = lax.broadcasted_iota(jnp.int32, (ENT * ES, N), 1)
    ent_oh = (iota == eidx).astype(f32)
    ent_h = jnp.dot(ent_oh, h, preferred_element_type=f32)

    m = ent_mask_ref[0]
    r_io = lax.broadcasted_iota(jnp.int32, (ENT, ENT * ES), 0)
    e_io = lax.broadcasted_iota(jnp.int32, (ENT, ENT * ES), 1)
    P = ((e_io // ES) == r_io).astype(f32) * m
    denom = jnp.sum(P, axis=1, keepdims=True) + 1e-6
    ent_rep = jnp.dot(P, ent_h, preferred_element_type=f32) / denom
    flat = jnp.concatenate([ent_rep[0:1, :], ent_rep[1:2, :]], axis=1)
    out_ref[0] = jnp.dot(flat, W_rel_ref[...],
                         preferred_element_type=f32) + b_rel_ref[...]


_sc_mesh = plsc.VectorSubcoreMesh(core_axis_name="c", subcore_axis_name="s")


@functools.partial(
    pl.kernel, mesh=_sc_mesh,
    out_type=jax.ShapeDtypeStruct((BG * 2 * N * H,), f32),
    scratch_types=[
        pltpu.VMEM((NCH, CH), jnp.int32),
        pltpu.VMEM((NBUF, CH, H), f32),
        pltpu.VMEM((N * H,), f32),
    ] + [pltpu.SemaphoreType.DMA] * NBUF,
)
def _sc_gather(t_hbm, gin_hbm, gout_hbm, rio_hbm, idx_v, gbuf, rep_v, *sems):
    wid = lax.axis_index("s") * _NC + lax.axis_index("c")
    bat = wid >> 1
    d_out = (wid & 1) == 1

    def gather(ci, bslot):
        pltpu.make_async_copy(t_hbm.at[idx_v.at[ci]], gbuf.at[bslot],
                              sems[bslot]).start()

    def gather_wait(ci, bslot):
        pltpu.make_async_copy(t_hbm.at[idx_v.at[ci]], gbuf.at[bslot],
                              sems[bslot]).wait()

    def accum(ci, bslot):
        def node_body(n, _):
            row0 = n * D
            for k in range(H // 16):
                vals = [gbuf[bslot, row0 + j, pl.ds(k * 16, 16)]
                        for j in range(D)]
                while len(vals) > 1:
                    vals = [vals[i] + vals[i + 1]
                            for i in range(0, len(vals), 2)]
                rep_v[pl.ds((ci * NODES_PER_CH + n) * H + k * 16, 16)] = vals[0]
            return 0
        lax.fori_loop(0, NODES_PER_CH, node_body, 0)

    def run_dir(idx_hbm):
        pltpu.sync_copy(idx_hbm.at[pl.ds(bat * NCH, NCH)], idx_v)
        for b in range(NBUF):
            gather(b, b)

        def group_body(g, _):
            for b in range(NBUF):
                ci = g * NBUF + b
                gather_wait(ci, b)
                accum(ci, b)
                nxt = ci + NBUF

                @pl.when(nxt < NCH)
                def _():
                    gather(nxt, b)
            return 0

        lax.fori_loop(0, NCH // NBUF, group_body, 0)
        pltpu.sync_copy(
            rep_v, rio_hbm.at[pl.ds((bat * 2 + (wid & 1)) * N * H, N * H)])

    @pl.when(jnp.logical_not(d_out))
    def _():
        run_dir(gin_hbm)

    @pl.when(d_out)
    def _():
        run_dir(gout_hbm)


def _bspec(blk, im):
    return pl.BlockSpec(blk, im)


def _full(arr):
    return pl.BlockSpec(arr.shape, lambda b: (0,) * arr.ndim)


def kernel(node_num, lemmas, lemmas_idx, lemmas_chars, in_nodes, in_labels,
           out_nodes, out_labels, entity_indexs, truth_tags,
           in_node_mask, out_node_mask, entity_mask,
           W_word, b_word, edge_emb, W_edge, b_edge,
           w_in_i, w_out_i, b_i, w_in_o, w_out_o, b_o,
           w_in_f, w_out_f, b_f, w_in_cell, w_out_cell, b_cell,
           W_rel, b_rel):
    We1 = W_edge[:H]
    We2 = W_edge[H:]
    Wcat = jnp.concatenate([
        jnp.concatenate([w_in_i, w_in_o, w_in_f, w_in_cell], axis=2),
        jnp.concatenate([w_out_i, w_out_o, w_out_f, w_out_cell], axis=2),
    ], axis=1)
    bcat = jnp.concatenate([b_i, b_o, b_f, b_cell], axis=1)
    b_word2 = b_word.reshape(1, H)
    b_edge2 = b_edge.reshape(1, H)
    ent_idx = entity_indexs.reshape(B, ENT * ES, 1).astype(jnp.int32)
    ent_m = entity_mask.reshape(B, 1, ENT * ES)

    nhw = lambda: _bspec((1, N, H), lambda b: (b, 0, 0))
    rio_spec = lambda: _bspec((1, 2, N, H), lambda b: (b, 0, 0, 0))

    def run_A(lem_g, inn_g, inl_g, outn_g, outl_g):
        return pl.pallas_call(
            _body_A,
            grid=(BG,),
            in_specs=[
                _bspec((1, N, WV), lambda b: (b, 0, 0)),
                _bspec((1, N, D), lambda b: (b, 0, 0)),
                _bspec((1, N, D), lambda b: (b, 0, 0)),
                _bspec((1, N, D), lambda b: (b, 0, 0)),
                _bspec((1, N, D), lambda b: (b, 0, 0)),
                _full(W_word),
                _bspec((1, H), lambda b: (0, 0)),
                _full(edge_emb),
                _full(We1),
                _full(We2),
                _bspec((1, H), lambda b: (0, 0)),
            ],
            out_specs=[
                nhw(),
                _bspec((1, EDIM, N, H), lambda b: (b, 0, 0, 0)),
                _bspec((1, N, D), lambda b: (b, 0, 0)),
                _bspec((1, N, D), lambda b: (b, 0, 0)),
            ],
            out_shape=[
                jax.ShapeDtypeStruct((BG, N, H), f32),
                jax.ShapeDtypeStruct((BG, EDIM, N, H), f32),
                jax.ShapeDtypeStruct((BG, N, D), jnp.int32),
                jax.ShapeDtypeStruct((BG, N, D), jnp.int32),
            ],
        )(lem_g, inn_g, inl_g, outn_g, outl_g,
          W_word, b_word2, edge_emb, We1, We2, b_edge2)

    def run_B(h, c, rio, l):
        return pl.pallas_call(
            _body_B,
            grid=(BG,),
            in_specs=[nhw(), nhw(), rio_spec(),
                      _full(Wcat[l]), _bspec((1, 4 * H), lambda b: (0, 0)),
                      _full(edge_emb), _full(We1), _full(We2),
                      _bspec((1, H), lambda b: (0, 0))],
            out_specs=[nhw(), nhw(),
                       _bspec((1, EDIM, N, H), lambda b: (b, 0, 0, 0))],
            out_shape=[
                jax.ShapeDtypeStruct((BG, N, H), f32),
                jax.ShapeDtypeStruct((BG, N, H), f32),
                jax.ShapeDtypeStruct((BG, EDIM, N, H), f32),
            ],
        )(h, c, rio, Wcat[l], bcat[l].reshape(1, 4 * H),
          edge_emb, We1, We2, b_edge2)

    def run_E(h, c, rio, eidx_g, em_g):
        return pl.pallas_call(
            _body_E,
            grid=(BG,),
            in_specs=[nhw(), nhw(), rio_spec(),
                      _full(Wcat[L - 1]), _bspec((1, 4 * H), lambda b: (0, 0)),
                      _bspec((1, ENT * ES, 1), lambda b: (b, 0, 0)),
                      _bspec((1, 1, ENT * ES), lambda b: (b, 0, 0)),
                      _full(W_rel), _bspec((1, REL), lambda b: (0, 0))],
            out_specs=_bspec((1, 1, REL), lambda b: (b, 0, 0)),
            out_shape=jax.ShapeDtypeStruct((BG, 1, REL), f32),
        )(h, c, rio, Wcat[L - 1], bcat[L - 1].reshape(1, 4 * H),
          eidx_g, em_g, W_rel, b_rel.reshape(1, REL))

    def run_group(sl):
        h, T, gin, gout = run_A(lemmas[sl], in_nodes[sl], in_labels[sl],
                                out_nodes[sl], out_labels[sl])
        gin2 = gin.reshape(BG * NCH, CH)
        gout2 = gout.reshape(BG * NCH, CH)
        c = jnp.zeros((BG, N, H), f32)
        for l in range(L - 1):
            rio = _sc_gather(T.reshape(BG * E_PB, H), gin2, gout2)
            h, c, T = run_B(h, c, rio.reshape(BG, 2, N, H), l)
        rio = _sc_gather(T.reshape(BG * E_PB, H), gin2, gout2)
        return run_E(h, c, rio.reshape(BG, 2, N, H), ent_idx[sl], ent_m[sl])

    out0 = run_group(slice(0, BG))
    out1 = run_group(slice(BG, B))
    return jnp.concatenate([out0, out1], axis=0).reshape(B, REL)
